# baseline (speedup 1.0000x reference)
import functools

import jax
import jax.numpy as jnp
from jax import lax
from jax.experimental import pallas as pl
from jax.experimental.pallas import tpu as pltpu
from jax.experimental.pallas import tpu_sc as plsc

_B = 4
_S = 2048
_H = 768
_NSL = _H // 16
_NW = 32
_TOK_PER_W = (_B * _S) // _NW
_CHUNK = 64
_SUB = 32
_NSUB = _TOK_PER_W // _SUB
_EPS = 1e-12


def _rsqrt16(v):
    i = lax.bitcast_convert_type(v, jnp.int32)
    y = lax.bitcast_convert_type(jnp.int32(0x5F3759DF) - (i >> 1),
                                 jnp.float32)
    for _ in range(3):
        y = y * (1.5 - 0.5 * v * y * y)
    return y


def _lanesum16(x):
    lanes = lax.iota(jnp.int32, 16)
    for off in (1, 2, 4, 8):
        x = x + x.at[lanes ^ off].get(mode="promise_in_bounds",
                                      unique_indices=True)
    return x


def _emb_body(ids_hbm, wt_hbm, pt_hbm, g_hbm, b_hbm, out_hbm,
              idx_v, rows_v, pos_v, g_v, b_v, sem_g0, sem_g1, sem_p):
    wid = lax.axis_index("s") * 2 + lax.axis_index("c")
    pos0 = wid * _CHUNK
    sems = (sem_g0, sem_g1)

    pltpu.sync_copy(g_hbm, g_v)
    pltpu.sync_copy(b_hbm, b_v)
    cp_p = pltpu.async_copy(pt_hbm.at[pl.ds(pos0, _CHUNK), :], pos_v, sem_p)

    def off_of(j):
        return (j >> 1) * _S + pos0 + (j & 1) * _SUB

    def fire(j, buf):
        pltpu.sync_copy(ids_hbm.at[pl.ds(off_of(j), _SUB)], idx_v.at[buf])
        pltpu.async_copy(wt_hbm.at[idx_v.at[buf]], rows_v.at[buf], sems[buf])

    def wait_g(buf):
        pltpu.make_async_copy(
            wt_hbm.at[idx_v.at[buf]], rows_v.at[buf], sems[buf]).wait()

    def compute(j, buf):
        rv = rows_v.at[buf]
        half = (j & 1) * _SUB

        def token_body(t, _):
            acc = jnp.zeros((16,), jnp.float32)
            acc2 = jnp.zeros((16,), jnp.float32)
            for i in range(_NSL):
                w = rv[t, pl.ds(i * 16, 16)]
                p = pos_v[half + t, pl.ds(i * 16, 16)]
                s = w + p
                rv[t, pl.ds(i * 16, 16)] = s
                acc = acc + s
                acc2 = acc2 + s * s
            m16 = _lanesum16(acc) * (1.0 / _H)
            var16 = _lanesum16(acc2) * (1.0 / _H) - m16 * m16
            k16 = _rsqrt16(var16 + _EPS)
            for i in range(_NSL):
                s = rv[t, pl.ds(i * 16, 16)]
                g = g_v[pl.ds(i * 16, 16)]
                b = b_v[pl.ds(i * 16, 16)]
                rv[t, pl.ds(i * 16, 16)] = (s - m16) * k16 * g + b
            return 0

        lax.fori_loop(0, _SUB, token_body, 0)
        pltpu.sync_copy(rv, out_hbm.at[pl.ds(off_of(j), _SUB), :])

    fire(0, 0)
    cp_p.wait()

    def outer(gidx, _):
        for buf in range(2):
            j = gidx * 2 + buf
            wait_g(buf)

            @pl.when(j + 1 < _NSUB)
            def _():
                fire(j + 1, buf ^ 1)

            compute(j, buf)
        return 0

    lax.fori_loop(0, _NSUB // 2, outer, 0)


def _sc_embeddings(ids_flat, word_table, pos_table, ln_gamma, ln_beta):
    mesh = plsc.VectorSubcoreMesh(core_axis_name="c", subcore_axis_name="s")
    f = functools.partial(
        pl.kernel,
        mesh=mesh,
        out_type=jax.ShapeDtypeStruct((_B * _S, _H), jnp.float32),
        scratch_types=[
            pltpu.VMEM((2, _SUB), jnp.int32),
            pltpu.VMEM((2, _SUB, _H), jnp.float32),
            pltpu.VMEM((_CHUNK, _H), jnp.float32),
            pltpu.VMEM((_H,), jnp.float32),
            pltpu.VMEM((_H,), jnp.float32),
            pltpu.SemaphoreType.DMA,
            pltpu.SemaphoreType.DMA,
            pltpu.SemaphoreType.DMA,
        ],
    )(_emb_body)
    return f(ids_flat, word_table, pos_table, ln_gamma, ln_beta)


_BLK_I = 256


def _mask_body(ids_ref, out_ref):
    i0 = pl.program_id(1) * _BLK_I
    ids = ids_ref[...]
    pad = jnp.broadcast_to(ids == 0, (1, _BLK_I, _S))
    ri = lax.broadcasted_iota(jnp.int32, (1, _BLK_I, _S), 1) + i0
    cj = lax.broadcasted_iota(jnp.int32, (1, _BLK_I, _S), 2)
    out_ref[...] = jnp.logical_or(pad, cj > ri)


def _tc_masks(input_ids):
    return pl.pallas_call(
        _mask_body,
        grid=(_B, _S // _BLK_I),
        in_specs=[pl.BlockSpec((1, 1, _S), lambda b, i: (b, 0, 0))],
        out_specs=pl.BlockSpec((1, _BLK_I, _S), lambda b, i: (b, i, 0)),
        out_shape=jax.ShapeDtypeStruct((_B, _S, _S), jnp.bool_),
    )(input_ids.reshape(_B, 1, _S))


def kernel(input_ids, word_table, pos_table, ln_gamma, ln_beta):
    ids_flat = input_ids.reshape(-1)
    emb = _sc_embeddings(ids_flat, word_table, pos_table, ln_gamma, ln_beta)
    masks = _tc_masks(input_ids)
    return emb.reshape(_B, _S, _H), masks


# speedup vs baseline: 1.1987x; 1.1987x over previous
import functools

import jax
import jax.numpy as jnp
from jax import lax
from jax.experimental import pallas as pl
from jax.experimental.pallas import tpu as pltpu
from jax.experimental.pallas import tpu_sc as plsc

_B = 4
_S = 2048
_H = 768
_NSL = _H // 16
_NW = 32
_TOK_PER_W = (_B * _S) // _NW
_CHUNK = 64
_SUB = 32
_NSUB = _TOK_PER_W // _SUB
_EPS = 1e-12


def _rsqrt16(v):
    i = lax.bitcast_convert_type(v, jnp.int32)
    y = lax.bitcast_convert_type(jnp.int32(0x5F3759DF) - (i >> 1),
                                 jnp.float32)
    for _ in range(3):
        y = y * (1.5 - 0.5 * v * y * y)
    return y


def _lanesum16(x):
    lanes = lax.iota(jnp.int32, 16)
    for off in (1, 2, 4, 8):
        x = x + x.at[lanes ^ off].get(mode="promise_in_bounds",
                                      unique_indices=True)
    return x


def _emb_body(ids_hbm, wt_hbm, pt_hbm, g_hbm, b_hbm, out_hbm,
              idx_v, rows_v, pos_v, g_v, b_v, sem_g0, sem_g1, sem_p):
    wid = lax.axis_index("s") * 2 + lax.axis_index("c")
    pos0 = wid * _CHUNK
    sems = (sem_g0, sem_g1)

    pltpu.sync_copy(g_hbm, g_v)
    pltpu.sync_copy(b_hbm, b_v)
    cp_p = pltpu.async_copy(pt_hbm.at[pl.ds(pos0, _CHUNK), :], pos_v, sem_p)

    def off_of(j):
        return (j >> 1) * _S + pos0 + (j & 1) * _SUB

    def fire(j, buf):
        pltpu.sync_copy(ids_hbm.at[pl.ds(off_of(j), _SUB)], idx_v.at[buf])
        pltpu.async_copy(wt_hbm.at[idx_v.at[buf]], rows_v.at[buf], sems[buf])

    def wait_g(buf):
        pltpu.make_async_copy(
            wt_hbm.at[idx_v.at[buf]], rows_v.at[buf], sems[buf]).wait()

    def compute(j, buf):
        rv = rows_v.at[buf]
        half = (j & 1) * _SUB

        @plsc.parallel_loop(0, _SUB, unroll=2)
        def token_body(t):
            row = rv.at[t]
            prow = pos_v.at[half + t]
            accs = [jnp.zeros((16,), jnp.float32) for _ in range(4)]
            accs2 = [jnp.zeros((16,), jnp.float32) for _ in range(4)]
            for i in range(_NSL):
                w = row[pl.ds(i * 16, 16)]
                p = prow[pl.ds(i * 16, 16)]
                s = w + p
                row[pl.ds(i * 16, 16)] = s
                accs[i % 4] = accs[i % 4] + s
                accs2[i % 4] = accs2[i % 4] + s * s
            acc = (accs[0] + accs[1]) + (accs[2] + accs[3])
            acc2 = (accs2[0] + accs2[1]) + (accs2[2] + accs2[3])
            m16 = _lanesum16(acc) * (1.0 / _H)
            var16 = _lanesum16(acc2) * (1.0 / _H) - m16 * m16
            k16 = _rsqrt16(var16 + _EPS)
            for i in range(_NSL):
                s = row[pl.ds(i * 16, 16)]
                g = g_v[pl.ds(i * 16, 16)]
                b = b_v[pl.ds(i * 16, 16)]
                row[pl.ds(i * 16, 16)] = (s - m16) * k16 * g + b
        pltpu.sync_copy(rv, out_hbm.at[pl.ds(off_of(j), _SUB), :])

    fire(0, 0)
    cp_p.wait()

    def outer(gidx, _):
        for buf in range(2):
            j = gidx * 2 + buf
            wait_g(buf)

            @pl.when(j + 1 < _NSUB)
            def _():
                fire(j + 1, buf ^ 1)

            compute(j, buf)
        return 0

    lax.fori_loop(0, _NSUB // 2, outer, 0)


def _sc_embeddings(ids_flat, word_table, pos_table, ln_gamma, ln_beta):
    mesh = plsc.VectorSubcoreMesh(core_axis_name="c", subcore_axis_name="s")
    f = functools.partial(
        pl.kernel,
        mesh=mesh,
        out_type=jax.ShapeDtypeStruct((_B * _S, _H), jnp.float32),
        scratch_types=[
            pltpu.VMEM((2, _SUB), jnp.int32),
            pltpu.VMEM((2, _SUB, _H), jnp.float32),
            pltpu.VMEM((_CHUNK, _H), jnp.float32),
            pltpu.VMEM((_H,), jnp.float32),
            pltpu.VMEM((_H,), jnp.float32),
            pltpu.SemaphoreType.DMA,
            pltpu.SemaphoreType.DMA,
            pltpu.SemaphoreType.DMA,
        ],
    )(_emb_body)
    return f(ids_flat, word_table, pos_table, ln_gamma, ln_beta)


_BLK_I = 256


def _mask_body(ids_ref, out_ref):
    i0 = pl.program_id(1) * _BLK_I
    ids = ids_ref[...]
    pad = jnp.broadcast_to(ids == 0, (1, _BLK_I, _S))
    ri = lax.broadcasted_iota(jnp.int32, (1, _BLK_I, _S), 1) + i0
    cj = lax.broadcasted_iota(jnp.int32, (1, _BLK_I, _S), 2)
    out_ref[...] = jnp.logical_or(pad, cj > ri)


def _tc_masks(input_ids):
    return pl.pallas_call(
        _mask_body,
        grid=(_B, _S // _BLK_I),
        in_specs=[pl.BlockSpec((1, 1, _S), lambda b, i: (b, 0, 0))],
        out_specs=pl.BlockSpec((1, _BLK_I, _S), lambda b, i: (b, i, 0)),
        out_shape=jax.ShapeDtypeStruct((_B, _S, _S), jnp.bool_),
    )(input_ids.reshape(_B, 1, _S))


def kernel(input_ids, word_table, pos_table, ln_gamma, ln_beta):
    ids_flat = input_ids.reshape(-1)
    emb = _sc_embeddings(ids_flat, word_table, pos_table, ln_gamma, ln_beta)
    masks = _tc_masks(input_ids)
    return emb.reshape(_B, _S, _H), masks
